# baseline (speedup 1.0000x reference)
import jax
import jax.numpy as jnp
from jax import lax
from jax.experimental import pallas as pl
from jax.experimental.pallas import tpu as pltpu
from jax.experimental.pallas import tpu_sc as plsc

_D = 32
_B = 16384
_L = 20

_NW = 32
_ROWS_W = _B // _NW
_C = 64
_NCHUNK = _ROWS_W // _C
_STEP = 128
_SPC = (_C * _L) // _STEP
_MSTEP = _ROWS_W // _STEP
_TOKS_W = _ROWS_W * _L


def _sc_body(mi_hbm, tok_hbm, mt_hbm, gt_hbm, out_hbm,
             mi_v, tok_v, t0_v, mrows, g0, g1, cb0, cb1,
             sem_m, sem_g0, sem_g1, sem_o0, sem_o1):
    wid = lax.axis_index("s") * 2 + lax.axis_index("c")
    base = wid * _ROWS_W

    pltpu.sync_copy(mi_hbm.at[wid], mi_v)
    pltpu.sync_copy(tok_hbm.at[wid], tok_v)
    pltpu.sync_copy(gt_hbm.at[0], t0_v)

    mdesc = [pltpu.async_copy(mt_hbm.at[mi_v.at[s]],
                              mrows.at[pl.ds(s * _STEP, _STEP)], sem_m)
             for s in range(_MSTEP)]

    gbufs, gsems = (g0, g1), (sem_g0, sem_g1)
    combs, osems = (cb0, cb1), (sem_o0, sem_o1)

    def fire(kk):
        buf = gbufs[kk % 2]
        return [pltpu.async_copy(
                    gt_hbm.at[tok_v.at[pl.ds(kk * _C * _L + s * _STEP, _STEP)]],
                    buf.at[pl.ds(s * _STEP, _STEP)], gsems[kk % 2])
                for s in range(_SPC)]

    pending = {0: fire(0)}
    odesc = [None, None]

    t0a = t0_v[pl.ds(0, 16)]
    t0b = t0_v[pl.ds(16, 16)]
    tail_mask = jax.lax.iota(jnp.int32, 16) >= (16 - (_L - 16))

    for kk in range(_NCHUNK):
        if kk + 1 < _NCHUNK:
            pending[kk + 1] = fire(kk + 1)
        for dsc in pending.pop(kk):
            dsc.wait()
        if kk % 2 == 0:
            mdesc[kk // 2].wait()
        buf = gbufs[kk % 2]
        comb = combs[kk % 2]
        if odesc[kk % 2] is not None:
            odesc[kk % 2].wait()

        def row_body(c, carry):
            gbase = c * _L
            acc0 = buf[gbase, pl.ds(0, 16)]
            acc1 = buf[gbase, pl.ds(16, 16)]
            for l in range(1, _L):
                acc0 = acc0 + buf[gbase + l, pl.ds(0, 16)]
                acc1 = acc1 + buf[gbase + l, pl.ds(16, 16)]
            tbase = kk * _C * _L + c * _L
            tok1 = tok_v[pl.ds(tbase, 16)]
            tok2 = tok_v[pl.ds(tbase + (_L - 16), 16)]
            nz = (plsc.all_reduce_population_count(tok1 == 0)
                  + plsc.all_reduce_population_count((tok2 == 0) & tail_mask))
            nzf = nz.astype(jnp.float32)
            inv = 1.0 / jnp.maximum(float(_L) - nzf, 1.0)
            comb[c, pl.ds(0, 16)] = mrows[kk * _C + c, pl.ds(0, 16)]
            comb[c, pl.ds(16, 16)] = mrows[kk * _C + c, pl.ds(16, 16)]
            comb[c, pl.ds(_D, 16)] = (acc0 - nzf * t0a) * inv
            comb[c, pl.ds(_D + 16, 16)] = (acc1 - nzf * t0b) * inv
            return carry
        lax.fori_loop(0, _C, row_body, 0)

        odesc[kk % 2] = pltpu.async_copy(
            comb, out_hbm.at[pl.ds(base + kk * _C, _C)], osems[kk % 2])

    for dsc in odesc:
        if dsc is not None:
            dsc.wait()


def kernel(movie_ids, genre_tokens, movie_table, genre_table):
    mi = movie_ids.astype(jnp.int32).reshape(_NW, _MSTEP, _STEP)
    tok = genre_tokens.astype(jnp.int32).reshape(_NW, _TOKS_W)
    mesh = plsc.VectorSubcoreMesh(core_axis_name="c", subcore_axis_name="s")
    run = pl.kernel(
        _sc_body,
        mesh=mesh,
        compiler_params=pltpu.CompilerParams(
            use_tc_tiling_on_sc=False, needs_layout_passes=False),
        out_type=jax.ShapeDtypeStruct((_B, 2 * _D), jnp.float32),
        scratch_types=[
            pltpu.VMEM((_MSTEP, _STEP), jnp.int32),
            pltpu.VMEM((_TOKS_W,), jnp.int32),
            pltpu.VMEM((_D,), jnp.float32),
            pltpu.VMEM((_ROWS_W, _D), jnp.float32),
            pltpu.VMEM((_C * _L, _D), jnp.float32),
            pltpu.VMEM((_C * _L, _D), jnp.float32),
            pltpu.VMEM((_C, 2 * _D), jnp.float32),
            pltpu.VMEM((_C, 2 * _D), jnp.float32),
            pltpu.SemaphoreType.DMA,
            pltpu.SemaphoreType.DMA,
            pltpu.SemaphoreType.DMA,
            pltpu.SemaphoreType.DMA,
            pltpu.SemaphoreType.DMA,
        ],
    )
    return run(mi, tok, movie_table, genre_table)
